# baseline (speedup 1.0000x reference)
import functools

import jax
import jax.numpy as jnp
from jax import lax
from jax.experimental import pallas as pl
from jax.experimental.pallas import tpu as pltpu
from jax.experimental.pallas import tpu_sc as plsc

_ZD = 256
_KD = 512
_NTOK = 784
_NPAD = 1024


def _conv2d_nhwc(x, w, b, stride, padding):
    y = lax.conv_general_dilated(
        x, w, (stride, stride), ((padding, padding), (padding, padding)),
        dimension_numbers=('NHWC', 'OIHW', 'NHWC'))
    return y + b


def _bn_relu_nhwc(x, g, bt, eps=1e-5):
    mean = jnp.mean(x, axis=(0, 1, 2), keepdims=True)
    var = jnp.mean((x - mean) ** 2, axis=(0, 1, 2), keepdims=True)
    return jax.nn.relu(g * (x - mean) / jnp.sqrt(var + eps) + bt)


def _conv_t2d_nchw(x, w, b, stride, padding):
    kh, kw = w.shape[2], w.shape[3]
    wt = jnp.transpose(w[:, :, ::-1, ::-1], (1, 0, 2, 3))
    ph = kh - 1 - padding
    pw = kw - 1 - padding
    if stride > 1:
        n, c, h, wd = x.shape
        xs = jnp.zeros((n, c, (h - 1) * stride + 1, (wd - 1) * stride + 1), x.dtype)
        xs = xs.at[:, :, ::stride, ::stride].set(x)
    else:
        xs = x
    y = lax.conv_general_dilated(
        xs, wt, (1, 1), ((ph, ph), (pw, pw)),
        dimension_numbers=('NCHW', 'OIHW', 'NCHW'))
    return y + b[None, :, None, None]


def _bn_relu_nchw(x, g, bt, eps=1e-5):
    mean = jnp.mean(x, axis=(0, 2, 3), keepdims=True)
    var = jnp.mean((x - mean) ** 2, axis=(0, 2, 3), keepdims=True)
    return jax.nn.relu(g[None, :, None, None] * (x - mean) / jnp.sqrt(var + eps)
                       + bt[None, :, None, None])


def _dist_body(h4_ref, e5w_ref, e5b_ref, w_ref, z_ref, j_ref):
    h4 = h4_ref[...]
    z = jnp.dot(h4, e5w_ref[...].T, preferred_element_type=jnp.float32)
    z = z + e5b_ref[...]
    z_ref[...] = z
    w = w_ref[...]
    zz = jnp.sum(z * z, axis=1, keepdims=True)
    ww = jnp.sum(w * w, axis=1)[None, :]
    s = jnp.dot(z, w.T, preferred_element_type=jnp.float32)
    d = zz - 2.0 * s + ww
    m = jnp.min(d, axis=1, keepdims=True)
    col = lax.broadcasted_iota(jnp.int32, d.shape, 1)
    j = jnp.min(jnp.where(d == m, col, _KD), axis=1)
    jp = jnp.concatenate([j, jnp.zeros((_NPAD - _NTOK,), jnp.int32)])
    j_ref[...] = jp.reshape(8, 128)


_SC_INFO = plsc.get_sparse_core_info()
_NW = _SC_INFO.num_cores * _SC_INFO.num_subcores
_BPW = _NPAD // _NW


def _sc_gather_body(table_hbm, idx_hbm, out_hbm, idx_v, rows_v, sem):
    wid = lax.axis_index("s") * _SC_INFO.num_cores + lax.axis_index("c")
    base = wid * _BPW
    pltpu.sync_copy(idx_hbm.at[pl.ds(base, _BPW)], idx_v)
    pltpu.async_copy(table_hbm.at[idx_v], rows_v, sem).wait()
    pltpu.sync_copy(rows_v, out_hbm.at[pl.ds(base, _BPW)])


_sc_gather = functools.partial(
    pl.kernel,
    mesh=plsc.VectorSubcoreMesh(core_axis_name="c", subcore_axis_name="s"),
    out_type=jax.ShapeDtypeStruct((_NPAD, _ZD), jnp.float32),
    scratch_types=[
        pltpu.VMEM((_BPW,), jnp.int32),
        pltpu.VMEM((_BPW, _ZD), jnp.float32),
        pltpu.SemaphoreType.DMA,
    ],
)(_sc_gather_body)


def _head_body(z_ref, wj_ref, d0w_ref, d0b_ref, d0g_ref, d0bt_ref,
               hd_ref, loss_ref):
    z = z_ref[...]
    wj = wj_ref[...]
    diff = z - wj
    loss_ref[0, 0] = jnp.sum(diff * diff) / _NTOK
    y = jnp.dot(wj, d0w_ref[...], preferred_element_type=jnp.float32)
    y = y + d0b_ref[...]
    mean = jnp.mean(y, axis=0, keepdims=True)
    var = jnp.mean((y - mean) ** 2, axis=0, keepdims=True)
    yn = d0g_ref[...] * (y - mean) / jnp.sqrt(var + 1e-5) + d0bt_ref[...]
    hd_ref[...] = jnp.maximum(yn, 0.0)


def _vq_stage(h4_flat, p):
    e5w = p['e5_w'].reshape(_ZD, 480)
    e5b = p['e5_b'].reshape(1, _ZD)
    z, j8 = pl.pallas_call(
        _dist_body,
        out_shape=[
            jax.ShapeDtypeStruct((_NTOK, _ZD), jnp.float32),
            jax.ShapeDtypeStruct((8, 128), jnp.int32),
        ],
    )(h4_flat, e5w, e5b, p['dictW'])
    wj = _sc_gather(p['dictW'], j8.reshape(_NPAD))[:_NTOK]
    d0w = p['d0_w'].reshape(_ZD, 480)
    d0b = p['d0_b'].reshape(1, 480)
    d0g = p['d0_g'].reshape(1, 480)
    d0bt = p['d0_bt'].reshape(1, 480)
    hd, loss = pl.pallas_call(
        _head_body,
        out_shape=[
            jax.ShapeDtypeStruct((_NTOK, 480), jnp.float32),
            jax.ShapeDtypeStruct((1, 1), jnp.float32),
        ],
        out_specs=[
            pl.BlockSpec(memory_space=pltpu.VMEM),
            pl.BlockSpec(memory_space=pltpu.SMEM),
        ],
    )(z, wj, d0w, d0b, d0g, d0bt)
    return hd, loss[0, 0]


def kernel(x, params):
    p = params
    h = jnp.transpose(x, (0, 2, 3, 1))
    h = _bn_relu_nhwc(_conv2d_nhwc(h, p['e0_w'], p['e0_b'], 1, 1),
                      p['e0_g'], p['e0_bt'])
    for i in range(4):
        h = _bn_relu_nhwc(
            _conv2d_nhwc(h, p['e%d_w' % (i + 1)], p['e%d_b' % (i + 1)], 2, 1),
            p['e%d_g' % (i + 1)], p['e%d_bt' % (i + 1)])
    n, hh, wwd, _ = h.shape
    h4 = h.reshape(n * hh * wwd, 480)
    hd0, loss = _vq_stage(h4, p)
    hd = jnp.transpose(hd0.reshape(n, hh, wwd, 480), (0, 3, 1, 2))
    for i in range(4):
        hd = _bn_relu_nchw(
            _conv_t2d_nchw(hd, p['d%d_w' % (i + 1)], p['d%d_b' % (i + 1)], 2, 1),
            p['d%d_g' % (i + 1)], p['d%d_bt' % (i + 1)])
    y = lax.conv_general_dilated(
        hd, p['d5_w'], (1, 1), ((1, 1), (1, 1)),
        dimension_numbers=('NCHW', 'OIHW', 'NCHW'))
    out = y + p['d5_b'][None, :, None, None]
    return out, loss, loss


# speedup vs baseline: 1.0001x; 1.0001x over previous
import functools

import jax
import jax.numpy as jnp
from jax import lax
from jax.experimental import pallas as pl
from jax.experimental.pallas import tpu as pltpu
from jax.experimental.pallas import tpu_sc as plsc

_ZD = 256
_KD = 512
_NTOK = 784
_NPAD = 1024


def _conv2d_nhwc(x, w, b, stride, padding):
    y = lax.conv_general_dilated(
        x, w, (stride, stride), ((padding, padding), (padding, padding)),
        dimension_numbers=('NHWC', 'OIHW', 'NHWC'))
    return y + b


def _bn_relu_nhwc(x, g, bt, eps=1e-5):
    mean = jnp.mean(x, axis=(0, 1, 2), keepdims=True)
    var = jnp.mean((x - mean) ** 2, axis=(0, 1, 2), keepdims=True)
    return jax.nn.relu(g * (x - mean) / jnp.sqrt(var + eps) + bt)


def _conv_t2d_ref(x, w, b, stride, padding):
    kh, kw = w.shape[2], w.shape[3]
    wt = jnp.transpose(w[:, :, ::-1, ::-1], (1, 0, 2, 3))
    ph = kh - 1 - padding
    pw = kw - 1 - padding
    if stride > 1:
        n, c, h, wd = x.shape
        xs = jnp.zeros((n, c, (h - 1) * stride + 1, (wd - 1) * stride + 1), x.dtype)
        xs = xs.at[:, :, ::stride, ::stride].set(x)
    else:
        xs = x
    y = lax.conv_general_dilated(
        xs, wt, (1, 1), ((ph, ph), (pw, pw)),
        dimension_numbers=('NCHW', 'OIHW', 'NCHW'))
    return y + b[None, :, None, None]


def _bn_relu_nchw(x, g, bt, eps=1e-5):
    mean = jnp.mean(x, axis=(0, 2, 3), keepdims=True)
    var = jnp.mean((x - mean) ** 2, axis=(0, 2, 3), keepdims=True)
    return jax.nn.relu(g[None, :, None, None] * (x - mean) / jnp.sqrt(var + eps)
                       + bt[None, :, None, None])


def _dist_body(h4_ref, e5w_ref, e5b_ref, w_ref, z_ref, j_ref):
    h4 = h4_ref[...]
    z = jnp.dot(h4, e5w_ref[...].T, preferred_element_type=jnp.float32)
    z = z + e5b_ref[...]
    z_ref[...] = z
    w = w_ref[...]
    zz = jnp.sum(z * z, axis=1, keepdims=True)
    ww = jnp.sum(w * w, axis=1)[None, :]
    s = jnp.dot(z, w.T, preferred_element_type=jnp.float32)
    d = zz - 2.0 * s + ww
    m = jnp.min(d, axis=1, keepdims=True)
    col = lax.broadcasted_iota(jnp.int32, d.shape, 1)
    j = jnp.min(jnp.where(d == m, col, _KD), axis=1)
    jp = jnp.concatenate([j, jnp.zeros((_NPAD - _NTOK,), jnp.int32)])
    j_ref[...] = jp.reshape(8, 128)


_SC_INFO = plsc.get_sparse_core_info()
_NW = _SC_INFO.num_cores * _SC_INFO.num_subcores
_BPW = _NPAD // _NW


def _sc_gather_body(table_hbm, idx_hbm, out_hbm, idx_v, rows_v, sem):
    wid = lax.axis_index("s") * _SC_INFO.num_cores + lax.axis_index("c")
    base = wid * _BPW
    pltpu.sync_copy(idx_hbm.at[pl.ds(base, _BPW)], idx_v)
    pltpu.async_copy(table_hbm.at[idx_v], rows_v, sem).wait()
    pltpu.sync_copy(rows_v, out_hbm.at[pl.ds(base, _BPW)])


_sc_gather = functools.partial(
    pl.kernel,
    mesh=plsc.VectorSubcoreMesh(core_axis_name="c", subcore_axis_name="s"),
    out_type=jax.ShapeDtypeStruct((_NPAD, _ZD), jnp.float32),
    scratch_types=[
        pltpu.VMEM((_BPW,), jnp.int32),
        pltpu.VMEM((_BPW, _ZD), jnp.float32),
        pltpu.SemaphoreType.DMA,
    ],
)(_sc_gather_body)


def _head_body(z_ref, wj_ref, d0w_ref, d0b_ref, d0g_ref, d0bt_ref,
               hd_ref, loss_ref):
    z = z_ref[...]
    wj = wj_ref[...]
    diff = z - wj
    loss_ref[0, 0] = jnp.sum(diff * diff) / _NTOK
    y = jnp.dot(wj, d0w_ref[...], preferred_element_type=jnp.float32)
    y = y + d0b_ref[...]
    mean = jnp.mean(y, axis=0, keepdims=True)
    var = jnp.mean((y - mean) ** 2, axis=0, keepdims=True)
    yn = d0g_ref[...] * (y - mean) / jnp.sqrt(var + 1e-5) + d0bt_ref[...]
    hd_ref[...] = jnp.maximum(yn, 0.0)


def _vq_stage(h4_flat, p):
    e5w = p['e5_w'].reshape(_ZD, 480)
    e5b = p['e5_b'].reshape(1, _ZD)
    z, j8 = pl.pallas_call(
        _dist_body,
        out_shape=[
            jax.ShapeDtypeStruct((_NTOK, _ZD), jnp.float32),
            jax.ShapeDtypeStruct((8, 128), jnp.int32),
        ],
    )(h4_flat, e5w, e5b, p['dictW'])
    wj = _sc_gather(p['dictW'], j8.reshape(_NPAD))[:_NTOK]
    d0w = p['d0_w'].reshape(_ZD, 480)
    d0b = p['d0_b'].reshape(1, 480)
    d0g = p['d0_g'].reshape(1, 480)
    d0bt = p['d0_bt'].reshape(1, 480)
    hd, loss = pl.pallas_call(
        _head_body,
        out_shape=[
            jax.ShapeDtypeStruct((_NTOK, 480), jnp.float32),
            jax.ShapeDtypeStruct((1, 1), jnp.float32),
        ],
        out_specs=[
            pl.BlockSpec(memory_space=pltpu.VMEM),
            pl.BlockSpec(memory_space=pltpu.SMEM),
        ],
    )(z, wj, d0w, d0b, d0g, d0bt)
    return hd, loss[0, 0]


def kernel(x, params):
    p = params
    h = jnp.transpose(x, (0, 2, 3, 1))
    h = _bn_relu_nhwc(_conv2d_nhwc(h, p['e0_w'], p['e0_b'], 1, 1),
                      p['e0_g'], p['e0_bt'])
    for i in range(4):
        h = _bn_relu_nhwc(
            _conv2d_nhwc(h, p['e%d_w' % (i + 1)], p['e%d_b' % (i + 1)], 2, 1),
            p['e%d_g' % (i + 1)], p['e%d_bt' % (i + 1)])
    n, hh, wwd, _ = h.shape
    h4 = h.reshape(n * hh * wwd, 480)
    hd0, loss = _vq_stage(h4, p)
    hd = jnp.transpose(hd0.reshape(n, hh, wwd, 480), (0, 3, 1, 2))
    for i in range(4):
        hd = _bn_relu_nchw(
            _conv_t2d_ref(hd, p['d%d_w' % (i + 1)], p['d%d_b' % (i + 1)], 2, 1),
            p['d%d_g' % (i + 1)], p['d%d_bt' % (i + 1)])
    y = lax.conv_general_dilated(
        hd, p['d5_w'], (1, 1), ((1, 1), (1, 1)),
        dimension_numbers=('NCHW', 'OIHW', 'NCHW'))
    out = y + p['d5_b'][None, :, None, None]
    return out, loss, loss
